# baseline (speedup 1.0000x reference)
import functools

import jax
import jax.numpy as jnp
from jax import lax
from jax.experimental import pallas as pl
from jax.experimental.pallas import tpu as pltpu
from jax.experimental.pallas import tpu_sc as plsc

N = 10000
E = 320000
D = 128
R = 8
B = 4

NC = 2
NS = 16
LANE = 16
KB = D // LANE
PHASES = KB // NC

NPAD = 10016
ROWS = NPAD * R
STRIPE = ROWS // NS
ZROWS = STRIPE // 8

CH = 512
NB = 4
NCH_T = 40
NGROUPS = NCH_T // NB
E_PAD = (NS * NCH_T + NB) * CH
GARBAGE = N * R


def _sc_body(h8, meta, q_out, acc, zb, mb0, mb1, mb2, mb3,
             r0, r1, r2, r3, sem_m, sem_g, sem_s, sem_z):
    mbs = [mb0, mb1, mb2, mb3]
    rows = [r0, r1, r2, r3]
    c = lax.axis_index("c")
    s = lax.axis_index("s")
    k = c * PHASES

    for i in range(ZROWS):
        zb[i, :] = jnp.zeros((LANE,), jnp.float32)

    stripe0 = s * STRIPE
    chunk0 = s * NCH_T

    def meta_slice(cj):
        return meta.at[:, pl.ds(pl.multiple_of(cj * CH, CH), CH)]

    def fire_meta(b, cj):
        pltpu.async_copy(meta_slice(cj), mbs[b], sem_m)

    def drain_meta(b, cj):
        pltpu.make_async_copy(meta_slice(cj), mbs[b], sem_m).wait()

    for p in range(PHASES):
        kb = k + p

        zd = [pltpu.async_copy(zb, acc.at[pl.ds(stripe0 + i * ZROWS, ZROWS)],
                               sem_z) for i in range(8)]
        for d in zd:
            d.wait()
        plsc.subcore_barrier()

        for b in range(NB):
            fire_meta(b, chunk0 + b)

        def group(g, _):
            jb = chunk0 + g * NB
            gd = []
            for b in range(NB):
                drain_meta(b, jb + b)
                for gg in range(CH // LANE):
                    sl = pl.ds(gg * LANE, LANE)
                    mbs[b][0, sl] = mbs[b][0, sl] + kb
                gd.append(pltpu.async_copy(h8.at[mbs[b].at[0]], rows[b], sem_g))
            sd = []
            for b in range(NB):
                gd[b].wait()
                sd.append(pltpu.async_copy(rows[b], acc.at[mbs[b].at[1]],
                                           sem_s, add=True))
            for b in range(NB):
                sd[b].wait()
                fire_meta(b, jb + NB + b)
            return 0

        lax.fori_loop(0, NGROUPS, group, 0)
        for b in range(NB):
            drain_meta(b, chunk0 + NCH_T + b)
        plsc.subcore_barrier()

        cd = []
        for i in range(8):
            rr0 = pl.ds(stripe0 + i * ZROWS, ZROWS)
            cd.append(pltpu.async_copy(acc.at[rr0], q_out.at[rr0, kb], sem_z))
        for d in cd:
            d.wait()
        plsc.subcore_barrier()


def _sc_pass(h8, meta):
    mesh = plsc.VectorSubcoreMesh(core_axis_name="c", subcore_axis_name="s")
    f = pl.kernel(
        _sc_body,
        out_type=jax.ShapeDtypeStruct((ROWS, KB, LANE), jnp.float32),
        mesh=mesh,
        scratch_types=(
            [pltpu.VMEM_SHARED((ROWS, LANE), jnp.float32)]
            + [pltpu.VMEM((ZROWS, LANE), jnp.float32)]
            + [pltpu.VMEM((2, CH), jnp.int32) for _ in range(NB)]
            + [pltpu.VMEM((CH, LANE), jnp.float32) for _ in range(NB)]
            + [pltpu.SemaphoreType.DMA] * 4
        ),
        compiler_params=pltpu.CompilerParams(use_tc_tiling_on_sc=False),
    )
    return f(h8, meta)


def _combine_body(apply_relu, q_ref, h_ref, coeff_ref, bcat_ref, wself_ref,
                  bias_ref, out_ref, wstack_ref):
    @pl.when(pl.program_id(0) == 0)
    def _build_wstack():
        for r in range(R):
            w = coeff_ref[r, 0] * bcat_ref[0:D, :]
            for b in range(1, B):
                w = w + coeff_ref[r, b] * bcat_ref[b * D:(b + 1) * D, :]
            wstack_ref[r * D:(r + 1) * D, :] = w

    msg = jnp.dot(q_ref[...], wstack_ref[...], preferred_element_type=jnp.float32,
                  precision=jax.lax.Precision.HIGHEST)
    slf = jnp.dot(h_ref[...], wself_ref[...], preferred_element_type=jnp.float32,
                  precision=jax.lax.Precision.HIGHEST)
    o = msg + slf + bias_ref[...]
    if apply_relu:
        o = jnp.maximum(o, 0.0)
    out_ref[...] = o


def _combine(q, h, coeff, bases_cat, wself, bias, apply_relu):
    bn = 1000
    grid = (N // bn,)
    return pl.pallas_call(
        functools.partial(_combine_body, apply_relu),
        grid=grid,
        in_specs=[
            pl.BlockSpec((bn, R * D), lambda i: (i, 0)),
            pl.BlockSpec((bn, D), lambda i: (i, 0)),
            pl.BlockSpec(memory_space=pltpu.SMEM),
            pl.BlockSpec((B * D, D), lambda i: (0, 0)),
            pl.BlockSpec((D, D), lambda i: (0, 0)),
            pl.BlockSpec((1, D), lambda i: (0, 0)),
        ],
        out_specs=pl.BlockSpec((bn, D), lambda i: (i, 0)),
        out_shape=jax.ShapeDtypeStruct((N, D), jnp.float32),
        scratch_shapes=[pltpu.VMEM((R * D, D), jnp.float32)],
    )(q, h, coeff, bases_cat, wself, bias)


def kernel(feats, edge_index, rel_types, coeff1, bases1, Wself1, bias1,
           coeff2, bases2, Wself2, bias2):
    src = edge_index[0]
    dst = edge_index[1]
    rel = rel_types

    idxg0 = src * KB
    idxs0 = dst * R + rel
    pad = E_PAD - E
    meta = jnp.stack([
        jnp.concatenate([idxg0, jnp.zeros((pad,), jnp.int32)]),
        jnp.concatenate([idxs0, jnp.full((pad,), GARBAGE, jnp.int32)]),
    ])

    q1 = _sc_pass(feats.reshape(N * KB, LANE), meta)
    h2 = _combine(q1.reshape(NPAD, R * D), feats, coeff1,
                  bases1.reshape(B * D, D), Wself1, bias1.reshape(1, D),
                  apply_relu=True)
    q2 = _sc_pass(h2.reshape(N * KB, LANE), meta)
    out = _combine(q2.reshape(NPAD, R * D), h2, coeff2,
                   bases2.reshape(B * D, D), Wself2, bias2.reshape(1, D),
                   apply_relu=False)
    return out


# speedup vs baseline: 1.4053x; 1.4053x over previous
import functools

import jax
import jax.numpy as jnp
from jax import lax
from jax.experimental import pallas as pl
from jax.experimental.pallas import tpu as pltpu
from jax.experimental.pallas import tpu_sc as plsc

N = 10000
E = 320000
D = 128
R = 8
B = 4

NC = 2
NS = 16
CB = 32
KB = D // CB
PHASES = KB // NC

NPAD = 10016
ROWS = NPAD * R
STRIPE = ROWS // NS
ZROWS = STRIPE // 8

CH = 256
NB = 4
NCH_T = 80
NGROUPS = NCH_T // NB
E_PAD = (NS * NCH_T + NB) * CH
GARBAGE = N * R


def _sc_body(h8, meta, q_out, acc, zb, mb0, mb1, mb2, mb3,
             r0, r1, r2, r3, sem_m, sem_g, sem_s, sem_z):
    mbs = [mb0, mb1, mb2, mb3]
    rows = [r0, r1, r2, r3]
    c = lax.axis_index("c")
    s = lax.axis_index("s")
    k = c * PHASES

    for i in range(ZROWS):
        zb[i, :] = jnp.zeros((CB,), jnp.bfloat16)

    stripe0 = s * STRIPE
    chunk0 = s * NCH_T

    def meta_slice(cj):
        return meta.at[:, pl.ds(pl.multiple_of(cj * CH, CH), CH)]

    def fire_meta(b, cj):
        pltpu.async_copy(meta_slice(cj), mbs[b], sem_m)

    def drain_meta(b, cj):
        pltpu.make_async_copy(meta_slice(cj), mbs[b], sem_m).wait()

    for p in range(PHASES):
        kb = k + p

        zd = [pltpu.async_copy(zb, acc.at[pl.ds(stripe0 + i * ZROWS, ZROWS)],
                               sem_z) for i in range(8)]
        for d in zd:
            d.wait()
        plsc.subcore_barrier()

        for b in range(NB):
            fire_meta(b, chunk0 + b)

        def group(g, _):
            jb = chunk0 + g * NB
            gd = []
            for b in range(NB):
                drain_meta(b, jb + b)
                for gg in range(CH // 16):
                    sl = pl.ds(gg * 16, 16)
                    mbs[b][0, sl] = mbs[b][0, sl] + kb
                gd.append(pltpu.async_copy(h8.at[mbs[b].at[0]], rows[b], sem_g))
            sd = []
            for b in range(NB):
                gd[b].wait()
                sd.append(pltpu.async_copy(rows[b], acc.at[mbs[b].at[1]],
                                           sem_s, add=True))
            for b in range(NB):
                sd[b].wait()
                fire_meta(b, jb + NB + b)
            return 0

        lax.fori_loop(0, NGROUPS, group, 0)
        for b in range(NB):
            drain_meta(b, chunk0 + NCH_T + b)
        plsc.subcore_barrier()

        cd = []
        for i in range(8):
            rr0 = pl.ds(stripe0 + i * ZROWS, ZROWS)
            cd.append(pltpu.async_copy(acc.at[rr0], q_out.at[rr0, kb], sem_z))
        for d in cd:
            d.wait()
        plsc.subcore_barrier()


def _sc_pass(h8, meta):
    mesh = plsc.VectorSubcoreMesh(core_axis_name="c", subcore_axis_name="s")
    f = pl.kernel(
        _sc_body,
        out_type=jax.ShapeDtypeStruct((ROWS, KB, CB), jnp.bfloat16),
        mesh=mesh,
        scratch_types=(
            [pltpu.VMEM_SHARED((ROWS, CB), jnp.bfloat16)]
            + [pltpu.VMEM((ZROWS, CB), jnp.bfloat16)]
            + [pltpu.VMEM((2, CH), jnp.int32) for _ in range(NB)]
            + [pltpu.VMEM((CH, CB), jnp.bfloat16) for _ in range(NB)]
            + [pltpu.SemaphoreType.DMA] * 4
        ),
        compiler_params=pltpu.CompilerParams(use_tc_tiling_on_sc=False),
    )
    return f(h8, meta)


def _combine_body(apply_relu, q_ref, h_ref, coeff_ref, bcat_ref, wself_ref,
                  bias_ref, out_ref, wstack_ref):
    @pl.when(pl.program_id(0) == 0)
    def _build_wstack():
        for r in range(R):
            w = coeff_ref[r, 0] * bcat_ref[0:D, :]
            for b in range(1, B):
                w = w + coeff_ref[r, b] * bcat_ref[b * D:(b + 1) * D, :]
            wstack_ref[r * D:(r + 1) * D, :] = w

    msg = jnp.dot(q_ref[...].astype(jnp.float32), wstack_ref[...],
                  preferred_element_type=jnp.float32,
                  precision=jax.lax.Precision.HIGHEST)
    slf = jnp.dot(h_ref[...], wself_ref[...], preferred_element_type=jnp.float32,
                  precision=jax.lax.Precision.HIGHEST)
    o = msg + slf + bias_ref[...]
    if apply_relu:
        o = jnp.maximum(o, 0.0)
    out_ref[...] = o


def _combine(q, h, coeff, bases_cat, wself, bias, apply_relu):
    bn = 1000
    grid = (N // bn,)
    return pl.pallas_call(
        functools.partial(_combine_body, apply_relu),
        grid=grid,
        in_specs=[
            pl.BlockSpec((bn, R * D), lambda i: (i, 0)),
            pl.BlockSpec((bn, D), lambda i: (i, 0)),
            pl.BlockSpec(memory_space=pltpu.SMEM),
            pl.BlockSpec((B * D, D), lambda i: (0, 0)),
            pl.BlockSpec((D, D), lambda i: (0, 0)),
            pl.BlockSpec((1, D), lambda i: (0, 0)),
        ],
        out_specs=pl.BlockSpec((bn, D), lambda i: (i, 0)),
        out_shape=jax.ShapeDtypeStruct((N, D), jnp.float32),
        scratch_shapes=[pltpu.VMEM((R * D, D), jnp.float32)],
    )(q, h, coeff, bases_cat, wself, bias)


def kernel(feats, edge_index, rel_types, coeff1, bases1, Wself1, bias1,
           coeff2, bases2, Wself2, bias2):
    src = edge_index[0]
    dst = edge_index[1]
    rel = rel_types

    idxg0 = src * KB
    idxs0 = dst * R + rel
    pad = E_PAD - E
    meta = jnp.stack([
        jnp.concatenate([idxg0, jnp.zeros((pad,), jnp.int32)]),
        jnp.concatenate([idxs0, jnp.full((pad,), GARBAGE, jnp.int32)]),
    ])

    q1 = _sc_pass(feats.astype(jnp.bfloat16).reshape(N * KB, CB), meta)
    h2 = _combine(q1.reshape(NPAD, R * D), feats, coeff1,
                  bases1.reshape(B * D, D), Wself1, bias1.reshape(1, D),
                  apply_relu=True)
    q2 = _sc_pass(h2.astype(jnp.bfloat16).reshape(N * KB, CB), meta)
    out = _combine(q2.reshape(NPAD, R * D), h2, coeff2,
                   bases2.reshape(B * D, D), Wself2, bias2.reshape(1, D),
                   apply_relu=False)
    return out


# speedup vs baseline: 1.4903x; 1.0605x over previous
import functools

import jax
import jax.numpy as jnp
from jax import lax
from jax.experimental import pallas as pl
from jax.experimental.pallas import tpu as pltpu
from jax.experimental.pallas import tpu_sc as plsc

N = 10000
E = 320000
D = 128
R = 8
B = 4

NC = 2
NS = 16
CB = 32
KB = D // CB
PHASES = KB // NC

NPAD = 10016
ROWS = NPAD * R
STRIPE = ROWS // NS
ZROWS = STRIPE // 8

CH = 256
NB = 4
NCH_T = 80
NGROUPS = NCH_T // NB
E_PAD = (NS * NCH_T + NB) * CH
GARBAGE = N * R


def _sc_body(h8, meta, q_out, acc, zb, mb0, mb1, mb2, mb3,
             r0, r1, r2, r3, sem_m, sem_g, sem_s, sem_z):
    mbs = [mb0, mb1, mb2, mb3]
    rows = [r0, r1, r2, r3]
    c = lax.axis_index("c")
    s = lax.axis_index("s")
    k = c * PHASES

    for i in range(ZROWS):
        zb[i, :] = jnp.zeros((CB,), jnp.bfloat16)

    stripe0 = s * STRIPE
    chunk0 = s * NCH_T

    def meta_slice(cj):
        return meta.at[:, pl.ds(pl.multiple_of(cj * CH, CH), CH)]

    def fire_meta(b, cj):
        pltpu.async_copy(meta_slice(cj), mbs[b], sem_m)

    def drain_meta(b, cj):
        pltpu.make_async_copy(meta_slice(cj), mbs[b], sem_m).wait()

    for p in range(PHASES):
        kb = k + p

        zd = [pltpu.async_copy(zb, acc.at[pl.ds(stripe0 + i * ZROWS, ZROWS)],
                               sem_z) for i in range(8)]
        for d in zd:
            d.wait()
        plsc.subcore_barrier()

        for b in range(NB):
            fire_meta(b, chunk0 + b)

        def group(g, _):
            jb = chunk0 + g * NB
            gd = []
            for b in range(NB):
                drain_meta(b, jb + b)
                for gg in range(CH // 16):
                    sl = pl.ds(gg * 16, 16)
                    mbs[b][0, sl] = mbs[b][0, sl] + kb
                gd.append(pltpu.async_copy(h8.at[mbs[b].at[0]], rows[b], sem_g))
            sd = []
            for b in range(NB):
                gd[b].wait()
                sd.append(pltpu.async_copy(rows[b], acc.at[mbs[b].at[1]],
                                           sem_s, add=True))
            for b in range(NB):
                sd[b].wait()
                fire_meta(b, jb + NB + b)
            return 0

        lax.fori_loop(0, NGROUPS, group, 0)
        for b in range(NB):
            drain_meta(b, chunk0 + NCH_T + b)
        plsc.subcore_barrier()

        cd = []
        for i in range(8):
            rr0 = pl.ds(stripe0 + i * ZROWS, ZROWS)
            cd.append(pltpu.async_copy(acc.at[rr0], q_out.at[rr0, kb], sem_z))
        for d in cd:
            d.wait()
        plsc.subcore_barrier()


def _sc_pass(h8, meta):
    mesh = plsc.VectorSubcoreMesh(core_axis_name="c", subcore_axis_name="s")
    f = pl.kernel(
        _sc_body,
        out_type=jax.ShapeDtypeStruct((ROWS, KB, CB), jnp.bfloat16),
        mesh=mesh,
        scratch_types=(
            [pltpu.VMEM_SHARED((ROWS, CB), jnp.bfloat16)]
            + [pltpu.VMEM((ZROWS, CB), jnp.bfloat16)]
            + [pltpu.VMEM((2, CH), jnp.int32) for _ in range(NB)]
            + [pltpu.VMEM((CH, CB), jnp.bfloat16) for _ in range(NB)]
            + [pltpu.SemaphoreType.DMA] * 4
        ),
        compiler_params=pltpu.CompilerParams(use_tc_tiling_on_sc=False),
    )
    return f(h8, meta)


def _combine_body(apply_relu, q_ref, h_ref, coeff_ref, bcat_ref, wself_ref,
                  bias_ref, out_ref, wstack_ref):
    @pl.when(pl.program_id(0) == 0)
    def _build_wstack():
        for r in range(R):
            w = coeff_ref[r, 0] * bcat_ref[0:D, :]
            for b in range(1, B):
                w = w + coeff_ref[r, b] * bcat_ref[b * D:(b + 1) * D, :]
            wstack_ref[r * D:(r + 1) * D, :] = w

    msg = jnp.dot(q_ref[...].astype(jnp.float32), wstack_ref[...],
                  preferred_element_type=jnp.float32,
                  precision=jax.lax.Precision.DEFAULT)
    slf = jnp.dot(h_ref[...], wself_ref[...], preferred_element_type=jnp.float32,
                  precision=jax.lax.Precision.HIGHEST)
    o = msg + slf + bias_ref[...]
    if apply_relu:
        o = jnp.maximum(o, 0.0)
    out_ref[...] = o


def _combine(q, h, coeff, bases_cat, wself, bias, apply_relu):
    bn = 1000
    grid = (N // bn,)
    return pl.pallas_call(
        functools.partial(_combine_body, apply_relu),
        grid=grid,
        in_specs=[
            pl.BlockSpec((bn, R * D), lambda i: (i, 0)),
            pl.BlockSpec((bn, D), lambda i: (i, 0)),
            pl.BlockSpec(memory_space=pltpu.SMEM),
            pl.BlockSpec((B * D, D), lambda i: (0, 0)),
            pl.BlockSpec((D, D), lambda i: (0, 0)),
            pl.BlockSpec((1, D), lambda i: (0, 0)),
        ],
        out_specs=pl.BlockSpec((bn, D), lambda i: (i, 0)),
        out_shape=jax.ShapeDtypeStruct((N, D), jnp.float32),
        scratch_shapes=[pltpu.VMEM((R * D, D), jnp.float32)],
    )(q, h, coeff, bases_cat, wself, bias)


def kernel(feats, edge_index, rel_types, coeff1, bases1, Wself1, bias1,
           coeff2, bases2, Wself2, bias2):
    src = edge_index[0]
    dst = edge_index[1]
    rel = rel_types

    idxg0 = src * KB
    idxs0 = dst * R + rel
    pad = E_PAD - E
    meta = jnp.stack([
        jnp.concatenate([idxg0, jnp.zeros((pad,), jnp.int32)]),
        jnp.concatenate([idxs0, jnp.full((pad,), GARBAGE, jnp.int32)]),
    ])

    q1 = _sc_pass(feats.astype(jnp.bfloat16).reshape(N * KB, CB), meta)
    h2 = _combine(q1.reshape(NPAD, R * D), feats, coeff1,
                  bases1.reshape(B * D, D), Wself1, bias1.reshape(1, D),
                  apply_relu=True)
    q2 = _sc_pass(h2.astype(jnp.bfloat16).reshape(N * KB, CB), meta)
    out = _combine(q2.reshape(NPAD, R * D), h2, coeff2,
                   bases2.reshape(B * D, D), Wself2, bias2.reshape(1, D),
                   apply_relu=False)
    return out
